# baseline (speedup 1.0000x reference)
import functools

import jax
import jax.numpy as jnp
from jax import lax
from jax.experimental import pallas as pl
from jax.experimental.pallas import tpu as pltpu
from jax.experimental.pallas import tpu_sc as plsc

D = 32
PD = 128
DIM = 64
K = 10
AVG_RATING = 3.5
CH = 128
G = 2


def _fm_sc(B, NR):
    info = plsc.get_sparse_core_info()
    NC, NS, L = info.num_cores, info.num_subcores, info.num_lanes
    NW = NC * NS
    BPW = B // NW
    NCHK = BPW // CH
    RPS = G * L
    SPC = CH // RPS

    mesh = plsc.VectorSubcoreMesh(core_axis_name="c", subcore_axis_name="s")

    @functools.partial(
        pl.kernel,
        out_type=jax.ShapeDtypeStruct((B,), jnp.float32),
        mesh=mesh,
        compiler_params=pltpu.CompilerParams(needs_layout_passes=False),
        scratch_types=[
            pltpu.VMEM((BPW,), jnp.int32),
            pltpu.VMEM((BPW,), jnp.int32),
            pltpu.VMEM((BPW,), jnp.int32),
            pltpu.VMEM((BPW,), jnp.int32),
            pltpu.VMEM((CH, PD), jnp.float32),
            pltpu.VMEM((CH, PD), jnp.float32),
            pltpu.VMEM((CH, PD), jnp.float32),
            pltpu.VMEM((CH, PD), jnp.float32),
            pltpu.VMEM((BPW,), jnp.float32),
            pltpu.VMEM((BPW,), jnp.float32),
            pltpu.VMEM(((DIM + 1) * 16,), jnp.float32),
            pltpu.SMEM(((DIM + 1) * 16,), jnp.float32),
            pltpu.VMEM((BPW,), jnp.float32),
            pltpu.SMEM((DIM,), jnp.float32),
            pltpu.SemaphoreType.DMA,
            pltpu.SemaphoreType.DMA,
            pltpu.SemaphoreType.DMA,
            pltpu.SemaphoreType.DMA,
            pltpu.SemaphoreType.DMA,
        ],
    )
    def body(user_h, item_h, uemb_h, iemb_h, bu_h, bi_h, prm_h, out_h,
             uidx_v, iidx_v, urow_v, irow_v, ub0, ub1, ib0, ib1, bu_v, bi_v,
             prm_v, prm_s, out_v, w2_s,
             sem_u0, sem_u1, sem_i0, sem_i1, sem_bias):
        wid = lax.axis_index("s") * NC + lax.axis_index("c")
        base = wid * BPW
        ubuf = (ub0, ub1)
        ibuf = (ib0, ib1)
        sem_u = (sem_u0, sem_u1)
        sem_i = (sem_i0, sem_i1)

        pltpu.sync_copy(user_h.at[pl.ds(base, BPW)], uidx_v)
        pltpu.sync_copy(item_h.at[pl.ds(base, BPW)], iidx_v)
        pltpu.sync_copy(prm_h, prm_v)

        for j in range(BPW // 16):
            r16 = pl.ds(j * 16, 16)
            urow_v[r16] = uidx_v[r16] >> 2
            irow_v[r16] = iidx_v[r16] >> 2

        bias_copies = []
        for j in range(NCHK):
            r = pl.ds(j * CH, CH)
            bias_copies.append(
                pltpu.async_copy(bu_h.at[uidx_v.at[r]], bu_v.at[r], sem_bias))
            bias_copies.append(
                pltpu.async_copy(bi_h.at[iidx_v.at[r]], bi_v.at[r], sem_bias))

        def issue(c, par):
            r = pl.ds(c * CH, CH)
            pltpu.async_copy(uemb_h.at[urow_v.at[r]], ubuf[par], sem_u[par])
            pltpu.async_copy(iemb_h.at[irow_v.at[r]], ibuf[par], sem_i[par])

        issue(0, 0)
        issue(1, 1)

        def prm_body(d, c):
            row = prm_v[pl.ds(d * 16, 16)]
            s = row[0] * row[0]
            prm_s[d * 16] = row[0]
            for k in range(1, K + 1):
                prm_s[d * 16 + k] = row[k]
                if k < K:
                    s = s + row[k] * row[k]
            w2_s[d] = s
            return c
        lax.fori_loop(0, DIM, prm_body, 0)
        rowb = prm_v[pl.ds(DIM * 16, 16)]
        prm_s[DIM * 16] = rowb[0]

        for cp in bias_copies:
            cp.wait()

        iota = lax.iota(jnp.int32, L)
        zero = jnp.zeros((L,), jnp.float32)
        fcb = prm_s[DIM * 16]

        def chunk_pair(t, carry):
            for par in range(2):
                c = 2 * t + par
                pltpu.make_async_copy(
                    uemb_h.at[pl.ds(0, CH)], ubuf[par], sem_u[par]).wait()
                pltpu.make_async_copy(
                    iemb_h.at[pl.ds(0, CH)], ibuf[par], sem_i[par]).wait()

                def step(t2, cy):
                    row0 = c * CH + t2 * RPS
                    loc0 = t2 * RPS
                    p = [[zero] * K for _ in range(G)]
                    lin = [zero] * G
                    sq = [zero] * G
                    slotv = [iota + (loc0 + g * L) for g in range(G)]
                    cbu = [(uidx_v[pl.ds(row0 + g * L, L)] & 3) * D
                           for g in range(G)]
                    cbi = [(iidx_v[pl.ds(row0 + g * L, L)] & 3) * D
                           for g in range(G)]
                    for d in range(DIM):
                        if d < D:
                            src, cb = ubuf[par], cbu
                        else:
                            src, cb = ibuf[par], cbi
                        w2d = w2_s[d]
                        wd = prm_s[d * 16 + K]
                        vs = [prm_s[d * 16 + k] for k in range(K)]
                        for g in range(G):
                            xd = plsc.load_gather(src, [slotv[g], cb[g] + (d % D)])
                            for k in range(K):
                                p[g][k] = p[g][k] + xd * vs[k]
                            lin[g] = lin[g] + xd * wd
                            sq[g] = sq[g] + (xd * xd) * w2d
                    for g in range(G):
                        f = p[g][0] * p[g][0]
                        for k in range(1, K):
                            f = f + p[g][k] * p[g][k]
                        r = pl.ds(row0 + g * L, L)
                        out_v[r] = (0.5 * (f - sq[g]) + lin[g]
                                    + bu_v[r] + bi_v[r] + fcb)
                    return cy
                lax.fori_loop(0, SPC, step, 0)

                @pl.when(c + 2 < NCHK)
                def _():
                    issue(c + 2, par)
            return carry
        lax.fori_loop(0, NCHK // 2, chunk_pair, 0)

        pltpu.sync_copy(out_v, out_h.at[pl.ds(base, BPW)])

    return body


def kernel(user, item, u_out, i_out, user_emb, item_emb, fc_W, fc_b, fm_V, b_users, b_items):
    B = user.shape[0]
    NB = user_emb.shape[0] // 8
    NR = user_emb.shape[0] * D // PD

    prm = jnp.zeros((DIM + 1, 16), jnp.float32)
    prm = prm.at[:DIM, :K].set(fm_V.astype(jnp.float32))
    prm = prm.at[:DIM, K].set(fc_W.astype(jnp.float32).reshape(DIM))
    prm = prm.at[DIM, 0].set(fc_b.astype(jnp.float32)[0] + AVG_RATING)

    return _fm_sc(B, NR)(
        user.astype(jnp.int32), item.astype(jnp.int32),
        user_emb.reshape(NR, PD), item_emb.reshape(NR, PD),
        b_users.reshape(-1), b_items.reshape(-1),
        prm.reshape(-1),
    )


# speedup vs baseline: 2.1095x; 2.1095x over previous
import functools

import jax
import jax.numpy as jnp
from jax import lax
from jax.experimental import pallas as pl
from jax.experimental.pallas import tpu as pltpu
from jax.experimental.pallas import tpu_sc as plsc

D = 32
PD = 128
DIM = 64
K = 10
AVG_RATING = 3.5
CH = 16
G = 1


def _fm_sc(B, NR):
    info = plsc.get_sparse_core_info()
    NC, NS, L = info.num_cores, info.num_subcores, info.num_lanes
    NW = NC * NS
    BPW = B // NW
    NCHK = BPW // CH
    RPS = G * L
    SPC = CH // RPS

    mesh = plsc.VectorSubcoreMesh(core_axis_name="c", subcore_axis_name="s")

    @functools.partial(
        pl.kernel,
        out_type=jax.ShapeDtypeStruct((B,), jnp.float32),
        mesh=mesh,
        compiler_params=pltpu.CompilerParams(
            needs_layout_passes=False, use_tc_tiling_on_sc=True),
        scratch_types=[
            pltpu.VMEM((BPW,), jnp.int32),
            pltpu.VMEM((BPW,), jnp.int32),
            pltpu.VMEM((CH, 8, D), jnp.float32),
            pltpu.VMEM((CH, 8, D), jnp.float32),
            pltpu.VMEM((CH, 8, D), jnp.float32),
            pltpu.VMEM((CH, 8, D), jnp.float32),
            pltpu.VMEM((BPW,), jnp.float32),
            pltpu.VMEM((BPW,), jnp.float32),
            pltpu.VMEM(((DIM + 1) * 16,), jnp.float32),
            pltpu.SMEM(((DIM + 1) * 16,), jnp.float32),
            pltpu.VMEM((BPW,), jnp.float32),
            pltpu.SMEM((DIM,), jnp.float32),
            pltpu.SemaphoreType.DMA,
            pltpu.SemaphoreType.DMA,
            pltpu.SemaphoreType.DMA,
            pltpu.SemaphoreType.DMA,
            pltpu.SemaphoreType.DMA,
        ],
    )
    def body(user_h, item_h, uemb_h, iemb_h, bu_h, bi_h, prm_h, out_h,
             uidx_v, iidx_v, ub0, ub1, ib0, ib1, bu_v, bi_v, prm_v, prm_s,
             out_v, w2_s, sem_u0, sem_u1, sem_i0, sem_i1, sem_bias):
        wid = lax.axis_index("s") * NC + lax.axis_index("c")
        base = wid * BPW

        ubuf = (ub0, ub1)
        ibuf = (ib0, ib1)
        sem_u = (sem_u0, sem_u1)
        sem_i = (sem_i0, sem_i1)

        pltpu.sync_copy(user_h.at[pl.ds(base, BPW)], uidx_v)
        pltpu.sync_copy(item_h.at[pl.ds(base, BPW)], iidx_v)
        pltpu.sync_copy(prm_h, prm_v)

        bias_copies = []
        for j in range(NCHK):
            r = pl.ds(j * CH, CH)
            bias_copies.append(
                pltpu.async_copy(bu_h.at[uidx_v.at[r]], bu_v.at[r], sem_bias))
            bias_copies.append(
                pltpu.async_copy(bi_h.at[iidx_v.at[r]], bi_v.at[r], sem_bias))

        def issue(c, par):
            uv = uidx_v[pl.ds(c * CH, L)] >> 3
            iv = iidx_v[pl.ds(c * CH, L)] >> 3
            for j in range(L):
                pltpu.async_copy(uemb_h.at[uv[j]], ubuf[par].at[j], sem_u[par])
                pltpu.async_copy(iemb_h.at[iv[j]], ibuf[par].at[j], sem_i[par])

        issue(0, 0)
        issue(1, 1)

        def prm_body(d, c):
            row = prm_v[pl.ds(d * 16, 16)]
            s = row[0] * row[0]
            prm_s[d * 16] = row[0]
            for k in range(1, K + 1):
                prm_s[d * 16 + k] = row[k]
                if k < K:
                    s = s + row[k] * row[k]
            w2_s[d] = s
            return c
        lax.fori_loop(0, DIM, prm_body, 0)
        rowb = prm_v[pl.ds(DIM * 16, 16)]
        prm_s[DIM * 16] = rowb[0]

        for cp in bias_copies:
            cp.wait()

        iota = lax.iota(jnp.int32, L)
        zero = jnp.zeros((L,), jnp.float32)
        fcb = prm_s[DIM * 16]

        def chunk_pair(t, carry):
            for par in range(2):
                c = 2 * t + par
                pltpu.make_async_copy(
                    uemb_h.at[pl.ds(0, CH)], ubuf[par], sem_u[par]).wait()
                pltpu.make_async_copy(
                    iemb_h.at[pl.ds(0, CH)], ibuf[par], sem_i[par]).wait()

                row0 = c * CH
                p = [zero] * K
                lin = zero
                sq = zero
                subu = uidx_v[pl.ds(row0, L)] & 7
                subi = iidx_v[pl.ds(row0, L)] & 7
                for d in range(DIM):
                    if d < D:
                        src, sub = ubuf[par], subu
                    else:
                        src, sub = ibuf[par], subi
                    colv = jnp.full((L,), d % D, jnp.int32)
                    w2d = w2_s[d]
                    wd = prm_s[d * 16 + K]
                    vs = [prm_s[d * 16 + k] for k in range(K)]
                    xd = plsc.load_gather(src, [iota, sub, colv])
                    for k in range(K):
                        p[k] = p[k] + xd * vs[k]
                    lin = lin + xd * wd
                    sq = sq + (xd * xd) * w2d
                f = p[0] * p[0]
                for k in range(1, K):
                    f = f + p[k] * p[k]
                r = pl.ds(row0, L)
                out_v[r] = (0.5 * (f - sq) + lin
                            + bu_v[r] + bi_v[r] + fcb)

                @pl.when(c + 2 < NCHK)
                def _():
                    issue(c + 2, par)
            return carry
        lax.fori_loop(0, NCHK // 2, chunk_pair, 0)

        pltpu.sync_copy(out_v, out_h.at[pl.ds(base, BPW)])

    return body


def kernel(user, item, u_out, i_out, user_emb, item_emb, fc_W, fc_b, fm_V, b_users, b_items):
    B = user.shape[0]
    NB = user_emb.shape[0] // 8
    NR = user_emb.shape[0] * D // PD

    prm = jnp.zeros((DIM + 1, 16), jnp.float32)
    prm = prm.at[:DIM, :K].set(fm_V.astype(jnp.float32))
    prm = prm.at[:DIM, K].set(fc_W.astype(jnp.float32).reshape(DIM))
    prm = prm.at[DIM, 0].set(fc_b.astype(jnp.float32)[0] + AVG_RATING)

    return _fm_sc(B, NB)(
        user.astype(jnp.int32), item.astype(jnp.int32),
        user_emb.reshape(NB, 8, D), item_emb.reshape(NB, 8, D),
        b_users.reshape(-1), b_items.reshape(-1),
        prm.reshape(-1),
    )
